# baseline (speedup 1.0000x reference)
import functools

import jax
import jax.numpy as jnp
from jax import lax
from jax.experimental import pallas as pl
from jax.experimental.pallas import tpu as pltpu
from jax.experimental.pallas import tpu_sc as plsc

_N = 10000
_E = 320000
_D = 128

_NC = 2
_NS = 16
_NW = _NC * _NS

_N_PAD = 10240
_HR = _N_PAD // 128
_ROWS_PER_TILE = _N_PAD // _NS
_EPT = _E // _NW
_EPT_PAD = 10240
_K = 128
_CH = _EPT_PAD // _K

_BLK = 512


def _deg_body(row_hbm, out_hbm, idxall_v, hist_v, rowid_v, zero_v, acc_sh, sem):
    cid = lax.axis_index("c")
    sid = lax.axis_index("s")
    wid = sid * _NC + cid

    zero16 = jnp.zeros((16,), jnp.float32)
    one16 = jnp.full((16,), 1.0, jnp.float32)
    for r in range(16):
        for c in range(8):
            zero_v[r, pl.ds(c * 16, 16)] = zero16
    for r in range(_HR):
        for c in range(8):
            hist_v[r, pl.ds(c * 16, 16)] = zero16
    for g in range(_HR // 16):
        rowid_v[pl.ds(g * 16, 16)] = lax.iota(jnp.int32, 16) + g * 16

    @pl.when(sid == 0)
    def _zero_acc():
        for j in range(_HR // 16):
            pltpu.sync_copy(zero_v, acc_sh.at[pl.ds(j * 16, 16)])

    pltpu.sync_copy(row_hbm.at[wid], idxall_v)
    plsc.subcore_barrier()

    for g in range(_EPT_PAD // 16):
        idx16 = idxall_v[pl.ds(g * 16, 16)]
        plsc.addupdate_scatter(hist_v, [idx16 >> 7, idx16 & 127], one16)

    pltpu.sync_copy(hist_v, acc_sh.at[rowid_v], add=True)
    plsc.subcore_barrier()

    @pl.when(sid == 0)
    def _copy_out():
        pltpu.sync_copy(acc_sh, out_hbm.at[cid])


_deg_call = functools.partial(
    pl.kernel,
    mesh=plsc.VectorSubcoreMesh(core_axis_name="c", subcore_axis_name="s"),
    compiler_params=pltpu.CompilerParams(needs_layout_passes=False),
    out_type=jax.ShapeDtypeStruct((_NC, _HR, 128), jnp.float32),
    scratch_types=[
        pltpu.VMEM((_EPT_PAD,), jnp.int32),
        pltpu.VMEM((_HR, 128), jnp.float32),
        pltpu.VMEM((_HR,), jnp.int32),
        pltpu.VMEM((16, 128), jnp.float32),
        pltpu.VMEM_SHARED((_HR, 128), jnp.float32),
        pltpu.SemaphoreType.DMA,
    ],
)(_deg_body)


def _agg_body(scaled_hbm, row_hbm, col_hbm, out_hbm,
              colall_v, ridxA, ridxB, rowsA, rowsB, zero_v, acc_sh,
              semA, semB):
    cid = lax.axis_index("c")
    sid = lax.axis_index("s")
    wid = sid * _NC + cid
    base = sid * _ROWS_PER_TILE

    zero16 = jnp.zeros((16,), jnp.float32)
    for r in range(16):
        for c in range(8):
            zero_v[r, pl.ds(c * 16, 16)] = zero16

    def zed(j, _):
        pltpu.sync_copy(zero_v, acc_sh.at[pl.ds(base + j * 16, 16)])
        return 0
    lax.fori_loop(0, _ROWS_PER_TILE // 16, zed, 0)

    pltpu.sync_copy(col_hbm.at[wid], colall_v)
    plsc.subcore_barrier()

    def body(i, _):
        pltpu.sync_copy(row_hbm.at[wid, i], ridxA)
        pltpu.async_copy(scaled_hbm.at[colall_v.at[i]], rowsA, semA).wait()
        pltpu.sync_copy(rowsA, acc_sh.at[ridxA], add=True)
        return 0
    lax.fori_loop(0, _CH, body, 0)

    plsc.subcore_barrier()
    pltpu.sync_copy(acc_sh.at[pl.ds(base, _ROWS_PER_TILE)],
                    out_hbm.at[cid, pl.ds(base, _ROWS_PER_TILE)])


_agg_call = functools.partial(
    pl.kernel,
    mesh=plsc.VectorSubcoreMesh(core_axis_name="c", subcore_axis_name="s"),
    compiler_params=pltpu.CompilerParams(needs_layout_passes=False),
    out_type=jax.ShapeDtypeStruct((_NC, _N_PAD, _D), jnp.float32),
    scratch_types=[
        pltpu.VMEM((_CH, _K), jnp.int32),
        pltpu.VMEM((_K,), jnp.int32),
        pltpu.VMEM((_K,), jnp.int32),
        pltpu.VMEM((_K, _D), jnp.float32),
        pltpu.VMEM((_K, _D), jnp.float32),
        pltpu.VMEM((16, _D), jnp.float32),
        pltpu.VMEM_SHARED((_N_PAD, _D), jnp.float32),
        pltpu.SemaphoreType.DMA,
        pltpu.SemaphoreType.DMA,
    ],
)(_agg_body)


def _dinv(deg_blk):
    deg = deg_blk[...]
    return jnp.where(deg > 0.0, lax.rsqrt(jnp.maximum(deg, 1.0)), 1.0)


def _linear_body(x_ref, wt_ref, b_ref, deg_ref, o_ref):
    y = jnp.dot(x_ref[...], wt_ref[...], preferred_element_type=jnp.float32)
    o_ref[...] = _dinv(deg_ref) * (y + b_ref[...])


_linear_call = pl.pallas_call(
    _linear_body,
    grid=(_N_PAD // _BLK,),
    in_specs=[
        pl.BlockSpec((_BLK, _D), lambda i: (i, 0)),
        pl.BlockSpec((_D, _D), lambda i: (0, 0)),
        pl.BlockSpec((1, _D), lambda i: (0, 0)),
        pl.BlockSpec((_BLK, 1), lambda i: (i, 0)),
    ],
    out_specs=pl.BlockSpec((_BLK, _D), lambda i: (i, 0)),
    out_shape=jax.ShapeDtypeStruct((_N_PAD, _D), jnp.float32),
)


def _combine_body(p_ref, deg_ref, o_ref):
    o_ref[...] = _dinv(deg_ref) * (p_ref[0] + p_ref[1])


_combine_call = pl.pallas_call(
    _combine_body,
    grid=(_N_PAD // _BLK,),
    in_specs=[
        pl.BlockSpec((_NC, _BLK, _D), lambda i: (0, i, 0)),
        pl.BlockSpec((_BLK, 1), lambda i: (i, 0)),
    ],
    out_specs=pl.BlockSpec((_BLK, _D), lambda i: (i, 0)),
    out_shape=jax.ShapeDtypeStruct((_N_PAD, _D), jnp.float32),
)


def kernel(node_features, edge_index, W, b):
    row2 = edge_index[0].reshape(_NW, _EPT)
    col2 = edge_index[1].reshape(_NW, _EPT)
    npad = _EPT_PAD - _EPT
    rowp = jnp.concatenate(
        [row2, jnp.full((_NW, npad), _N_PAD - 1, jnp.int32)], axis=1)
    colp = jnp.concatenate(
        [col2, jnp.zeros((_NW, npad), jnp.int32)], axis=1)
    row3 = rowp.reshape(_NW, _CH, _K)
    col3 = colp.reshape(_NW, _CH, _K)
    x_pad = jnp.concatenate(
        [node_features, jnp.zeros((_N_PAD - _N, _D), jnp.float32)], axis=0)
    degp = _deg_call(rowp)
    deg = (degp[0] + degp[1]).reshape(_N_PAD, 1)
    scaled = _linear_call(x_pad, W.T, b.reshape(1, _D), deg)
    partial = _agg_call(scaled, row3, col3)
    out = _combine_call(partial, deg)
    return out[:_N]


# speedup vs baseline: 1.6706x; 1.6706x over previous
import functools

import jax
import jax.numpy as jnp
from jax import lax
from jax.experimental import pallas as pl
from jax.experimental.pallas import tpu as pltpu
from jax.experimental.pallas import tpu_sc as plsc

_N = 10000
_E = 320000
_D = 128

_NC = 2
_NS = 16
_NW = _NC * _NS

_N_PAD = 10240
_HR = _N_PAD // 128
_ROWS_PER_TILE = _N_PAD // _NS
_EPT = _E // _NW
_EPT_PAD = 10240
_K = 80
_ITERS = _EPT // _K

_BLK = 512


def _deg_body(row_hbm, out_hbm, idxall_v, hist_v, rowid_v, zero_v, acc_sh, sem):
    cid = lax.axis_index("c")
    sid = lax.axis_index("s")
    wid = sid * _NC + cid

    zero16 = jnp.zeros((16,), jnp.float32)
    one16 = jnp.full((16,), 1.0, jnp.float32)
    for r in range(16):
        for c in range(8):
            zero_v[r, pl.ds(c * 16, 16)] = zero16
    for r in range(_HR):
        for c in range(8):
            hist_v[r, pl.ds(c * 16, 16)] = zero16
    for g in range(_HR // 16):
        rowid_v[pl.ds(g * 16, 16)] = lax.iota(jnp.int32, 16) + g * 16

    @pl.when(sid == 0)
    def _zero_acc():
        for j in range(_HR // 16):
            pltpu.sync_copy(zero_v, acc_sh.at[pl.ds(j * 16, 16)])

    pltpu.sync_copy(row_hbm.at[wid], idxall_v)
    plsc.subcore_barrier()

    for g in range(_EPT_PAD // 16):
        idx16 = idxall_v[pl.ds(g * 16, 16)]
        plsc.addupdate_scatter(hist_v, [idx16 >> 7, idx16 & 127], one16)

    pltpu.sync_copy(hist_v, acc_sh.at[rowid_v], add=True)
    plsc.subcore_barrier()

    @pl.when(sid == 0)
    def _copy_out():
        pltpu.sync_copy(acc_sh, out_hbm.at[cid])


_deg_call = functools.partial(
    pl.kernel,
    mesh=plsc.VectorSubcoreMesh(core_axis_name="c", subcore_axis_name="s"),
    compiler_params=pltpu.CompilerParams(needs_layout_passes=False),
    out_type=jax.ShapeDtypeStruct((_NC, _HR, 128), jnp.float32),
    scratch_types=[
        pltpu.VMEM((_EPT_PAD,), jnp.int32),
        pltpu.VMEM((_HR, 128), jnp.float32),
        pltpu.VMEM((_HR,), jnp.int32),
        pltpu.VMEM((16, 128), jnp.float32),
        pltpu.VMEM_SHARED((_HR, 128), jnp.float32),
        pltpu.SemaphoreType.DMA,
    ],
)(_deg_body)


def _agg_body(scaled_hbm, row_hbm, col_hbm, out_hbm,
              ridx_v, cidx_v, rows_v, zero_v, acc_sh, sem):
    cid = lax.axis_index("c")
    sid = lax.axis_index("s")
    wid = sid * _NC + cid
    base = sid * _ROWS_PER_TILE

    zero16 = jnp.zeros((16,), jnp.float32)
    for r in range(16):
        for c in range(8):
            zero_v[r, pl.ds(c * 16, 16)] = zero16

    def zed(j, _):
        pltpu.sync_copy(zero_v, acc_sh.at[pl.ds(base + j * 16, 16)])
        return 0
    lax.fori_loop(0, _ROWS_PER_TILE // 16, zed, 0)
    plsc.subcore_barrier()

    def body(i, _):
        pltpu.sync_copy(col_hbm.at[wid, i], cidx_v)
        gather = pltpu.async_copy(scaled_hbm.at[cidx_v], rows_v, sem)
        pltpu.sync_copy(row_hbm.at[wid, i], ridx_v)
        gather.wait()
        pltpu.sync_copy(rows_v, acc_sh.at[ridx_v], add=True)
        return 0
    lax.fori_loop(0, _ITERS, body, 0)

    plsc.subcore_barrier()
    pltpu.sync_copy(acc_sh.at[pl.ds(base, _ROWS_PER_TILE)],
                    out_hbm.at[cid, pl.ds(base, _ROWS_PER_TILE)])


_agg_call = functools.partial(
    pl.kernel,
    mesh=plsc.VectorSubcoreMesh(core_axis_name="c", subcore_axis_name="s"),
    compiler_params=pltpu.CompilerParams(needs_layout_passes=False),
    out_type=jax.ShapeDtypeStruct((_NC, _N_PAD, _D), jnp.float32),
    scratch_types=[
        pltpu.VMEM((_K,), jnp.int32),
        pltpu.VMEM((_K,), jnp.int32),
        pltpu.VMEM((_K, _D), jnp.float32),
        pltpu.VMEM((16, _D), jnp.float32),
        pltpu.VMEM_SHARED((_N_PAD, _D), jnp.float32),
        pltpu.SemaphoreType.DMA,
    ],
)(_agg_body)


def _dinv(deg_blk):
    deg = deg_blk[...]
    return jnp.where(deg > 0.0, lax.rsqrt(jnp.maximum(deg, 1.0)), 1.0)


def _linear_body(x_ref, wt_ref, b_ref, deg_ref, o_ref):
    y = jnp.dot(x_ref[...], wt_ref[...], preferred_element_type=jnp.float32)
    o_ref[...] = _dinv(deg_ref) * (y + b_ref[...])


_linear_call = pl.pallas_call(
    _linear_body,
    grid=(_N_PAD // _BLK,),
    in_specs=[
        pl.BlockSpec((_BLK, _D), lambda i: (i, 0)),
        pl.BlockSpec((_D, _D), lambda i: (0, 0)),
        pl.BlockSpec((1, _D), lambda i: (0, 0)),
        pl.BlockSpec((_BLK, 1), lambda i: (i, 0)),
    ],
    out_specs=pl.BlockSpec((_BLK, _D), lambda i: (i, 0)),
    out_shape=jax.ShapeDtypeStruct((_N_PAD, _D), jnp.float32),
)


def _combine_body(p_ref, deg_ref, o_ref):
    o_ref[...] = _dinv(deg_ref) * (p_ref[0] + p_ref[1])


_combine_call = pl.pallas_call(
    _combine_body,
    grid=(_N_PAD // _BLK,),
    in_specs=[
        pl.BlockSpec((_NC, _BLK, _D), lambda i: (0, i, 0)),
        pl.BlockSpec((_BLK, 1), lambda i: (i, 0)),
    ],
    out_specs=pl.BlockSpec((_BLK, _D), lambda i: (i, 0)),
    out_shape=jax.ShapeDtypeStruct((_N_PAD, _D), jnp.float32),
)


def kernel(node_features, edge_index, W, b):
    row2 = edge_index[0].reshape(_NW, _EPT)
    col2 = edge_index[1].reshape(_NW, _EPT)
    npad = _EPT_PAD - _EPT
    rowp = jnp.concatenate(
        [row2, jnp.full((_NW, npad), _N_PAD - 1, jnp.int32)], axis=1)
    row3 = row2.reshape(_NW, _ITERS, _K)
    col3 = col2.reshape(_NW, _ITERS, _K)
    x_pad = jnp.concatenate(
        [node_features, jnp.zeros((_N_PAD - _N, _D), jnp.float32)], axis=0)
    degp = _deg_call(rowp)
    deg = (degp[0] + degp[1]).reshape(_N_PAD, 1)
    scaled = _linear_call(x_pad, W.T, b.reshape(1, _D), deg)
    partial = _agg_call(scaled, row3, col3)
    out = _combine_call(partial, deg)
    return out[:_N]
